# baseline (speedup 1.0000x reference)
import functools

import jax
import jax.numpy as jnp
from jax import lax
from jax.experimental import pallas as pl
from jax.experimental.pallas import tpu as pltpu
from jax.experimental.pallas import tpu_sc as plsc

N_NODES = 10000
N_EDGES = 320000
D = 128

NC = 2
NS = 16

CH = 128
WIN = 16
NWIN = 10
NCH = WIN * NWIN
EPT = NCH * CH
EPAD = NS * EPT
TRASH = 10239

NPAD = 10240
RPT = NPAD // NS
RCOPIES = RPT // CH


def _sc_body(x_ref, src_ref, dst_ref, agg_out, cnt_out,
             idxs_v, idxd_v, buf_a, buf_b, acc_sh, sem_a, sem_b):
    cid = lax.axis_index("c")
    sid = lax.axis_index("s")

    zeros16 = jnp.zeros((16,), jnp.float32)
    one16 = jnp.full((16,), 1.0, jnp.float32)

    def fill_zero(i, carry):
        for c in range(D // 16):
            buf_b[i, pl.ds(c * 16, 16)] = zeros16
        return carry

    lax.fori_loop(0, CH, fill_zero, 0)

    def zero_acc(j, carry):
        pltpu.sync_copy(buf_b, acc_sh.at[pl.ds(sid * RPT + j * CH, CH), :])
        return carry

    lax.fori_loop(0, RCOPIES, zero_acc, 0)

    @pl.when(cid == 1)
    def _():
        col0 = jnp.where(lax.iota(jnp.int32, 16) == 0, one16, zeros16)

        def fill_cnt(i, carry):
            for c in range(D // 16):
                buf_a[i, pl.ds(c * 16, 16)] = zeros16
            buf_a[i, pl.ds(0, 16)] = col0
            return carry

        lax.fori_loop(0, CH, fill_cnt, 0)

    plsc.subcore_barrier()

    @pl.when(cid == 0)
    def _():
        def window(w, carry):
            pltpu.sync_copy(src_ref.at[sid, pl.ds(w * WIN, WIN)], idxs_v)
            pltpu.sync_copy(dst_ref.at[sid, pl.ds(w * WIN, WIN)], idxd_v)
            pltpu.async_copy(x_ref.at[idxs_v.at[0]], buf_a, sem_a)

            def pair(j, c2):
                c0 = 2 * j
                c1 = 2 * j + 1
                pltpu.async_copy(x_ref.at[idxs_v.at[c1]], buf_b, sem_b)
                pltpu.make_async_copy(
                    x_ref.at[idxs_v.at[c0]], buf_a, sem_a).wait()
                pltpu.sync_copy(buf_a, acc_sh.at[idxd_v.at[c0]], add=True)

                @pl.when(j < WIN // 2 - 1)
                def _():
                    pltpu.async_copy(x_ref.at[idxs_v.at[c1 + 1]], buf_a, sem_a)

                pltpu.make_async_copy(
                    x_ref.at[idxs_v.at[c1]], buf_b, sem_b).wait()
                pltpu.sync_copy(buf_b, acc_sh.at[idxd_v.at[c1]], add=True)
                return c2

            lax.fori_loop(0, WIN // 2, pair, 0)
            return carry

        lax.fori_loop(0, NWIN, window, 0)

    @pl.when(cid == 1)
    def _():
        def window(w, carry):
            pltpu.sync_copy(dst_ref.at[sid, pl.ds(w * WIN, WIN)], idxd_v)

            def cchunk(i, c2):
                pltpu.sync_copy(buf_a, acc_sh.at[idxd_v.at[i]], add=True)
                return c2

            lax.fori_loop(0, WIN, cchunk, 0)
            return carry

        lax.fori_loop(0, NWIN, window, 0)

    plsc.subcore_barrier()

    @pl.when(cid == 0)
    def _():
        def out_copy(j, carry):
            base = sid * RPT + j * CH
            pltpu.sync_copy(acc_sh.at[pl.ds(base, CH), :],
                            agg_out.at[pl.ds(base, CH), :])
            return carry
        lax.fori_loop(0, RCOPIES, out_copy, 0)

    @pl.when(cid == 1)
    def _():
        def out_copy(j, carry):
            base = sid * RPT + j * CH
            pltpu.sync_copy(acc_sh.at[pl.ds(base, CH), :],
                            cnt_out.at[pl.ds(base, CH), :])
            return carry
        lax.fori_loop(0, RCOPIES, out_copy, 0)


@functools.lru_cache(maxsize=1)
def _sc_agg():
    return functools.partial(
        pl.kernel,
        mesh=plsc.VectorSubcoreMesh(core_axis_name="c", subcore_axis_name="s",
                                    num_cores=NC, num_subcores=NS),
        out_type=(
            jax.ShapeDtypeStruct((NPAD, D), jnp.float32),
            jax.ShapeDtypeStruct((NPAD, D), jnp.float32),
        ),
        scratch_types=[
            pltpu.VMEM((WIN, CH), jnp.int32),
            pltpu.VMEM((WIN, CH), jnp.int32),
            pltpu.VMEM((CH, D), jnp.float32),
            pltpu.VMEM((CH, D), jnp.float32),
            pltpu.VMEM_SHARED((NPAD, D), jnp.float32),
            pltpu.SemaphoreType.DMA,
            pltpu.SemaphoreType.DMA,
        ],
    )(_sc_body)


BM = 1000


def _tc_body(p_ref, c_ref, x_ref, w_ref, b_ref, o_ref):
    cnt = c_ref[:, 0:1]
    inv = jnp.where(cnt > 0, 1.0 / cnt, 0.0)
    h = p_ref[...] * inv
    dn = (((1,), (1,)), ((), ()))
    out = lax.dot_general(h, w_ref[:, 0:D], dn,
                          preferred_element_type=jnp.float32,
                          precision=lax.Precision.HIGHEST)
    out += lax.dot_general(x_ref[...], w_ref[:, D:2 * D], dn,
                           preferred_element_type=jnp.float32,
                           precision=lax.Precision.HIGHEST)
    o_ref[...] = out + b_ref[...]


def _tc_linear(p, c, x, W, b2):
    return pl.pallas_call(
        _tc_body,
        grid=(N_NODES // BM,),
        in_specs=[
            pl.BlockSpec((BM, D), lambda m: (m, 0)),
            pl.BlockSpec((BM, D), lambda m: (m, 0)),
            pl.BlockSpec((BM, D), lambda m: (m, 0)),
            pl.BlockSpec((D, 2 * D), lambda m: (0, 0)),
            pl.BlockSpec((1, D), lambda m: (0, 0)),
        ],
        out_specs=pl.BlockSpec((BM, D), lambda m: (m, 0)),
        out_shape=jax.ShapeDtypeStruct((N_NODES, D), jnp.float32),
    )(p, c, x, W, b2)


def kernel(x, edge_index, W, b):
    src = edge_index[0].astype(jnp.int32)
    dst = edge_index[1].astype(jnp.int32)
    pad = EPAD - N_EDGES
    srcp = jnp.concatenate([src, jnp.zeros((pad,), jnp.int32)]
                           ).reshape(NS, NCH, CH)
    dstp = jnp.concatenate([dst, jnp.full((pad,), TRASH, jnp.int32)]
                           ).reshape(NS, NCH, CH)
    p, c = _sc_agg()(x, srcp, dstp)
    b2 = b.reshape(1, D)
    return _tc_linear(p, c, x, W, b2)


# speedup vs baseline: 1.0852x; 1.0852x over previous
import functools

import jax
import jax.numpy as jnp
from jax import lax
from jax.experimental import pallas as pl
from jax.experimental.pallas import tpu as pltpu
from jax.experimental.pallas import tpu_sc as plsc

N_NODES = 10000
N_EDGES = 320000
D = 128

NC = 2
NS = 16
NW = NC * NS

CH = 128
WIN = 16
NWIN = 5
NCH = WIN * NWIN
EPT = NCH * CH
EPAD = NW * EPT
TRASH = 10239

NPAD = 10240
RPT = NPAD // NS
RCOPIES = RPT // CH


def _sc_body(x_ref, src_ref, dst_ref, ones_ref, agg_out, cnt_out,
             idxs_v, idxd_v, buf_a, buf_b, acc_sh, sem_a, sem_b):
    cid = lax.axis_index("c")
    sid = lax.axis_index("s")
    wid = cid * NS + sid

    zeros16 = jnp.zeros((16,), jnp.float32)
    one16 = jnp.full((16,), 1.0, jnp.float32)

    def fill_zero(i, carry):
        for c in range(D // 16):
            buf_b[i, pl.ds(c * 16, 16)] = zeros16
        return carry

    lax.fori_loop(0, CH, fill_zero, 0)

    def zero_acc(j, carry):
        pltpu.sync_copy(buf_b, acc_sh.at[pl.ds(sid * RPT + j * CH, CH), :])
        return carry

    lax.fori_loop(0, RCOPIES, zero_acc, 0)
    plsc.subcore_barrier()

    def window1(w, carry):
        pltpu.sync_copy(src_ref.at[wid, pl.ds(w * WIN, WIN)], idxs_v)
        pltpu.sync_copy(dst_ref.at[wid, pl.ds(w * WIN, WIN)], idxd_v)
        pltpu.async_copy(x_ref.at[idxs_v.at[0]], buf_a, sem_a)

        def pair(j, c2):
            c0 = 2 * j
            c1 = 2 * j + 1
            pltpu.async_copy(x_ref.at[idxs_v.at[c1]], buf_b, sem_b)
            pltpu.make_async_copy(
                x_ref.at[idxs_v.at[c0]], buf_a, sem_a).wait()
            pltpu.sync_copy(buf_a, acc_sh.at[idxd_v.at[c0]], add=True)

            @pl.when(j < WIN // 2 - 1)
            def _():
                pltpu.async_copy(x_ref.at[idxs_v.at[c1 + 1]], buf_a, sem_a)

            pltpu.make_async_copy(
                x_ref.at[idxs_v.at[c1]], buf_b, sem_b).wait()
            pltpu.sync_copy(buf_b, acc_sh.at[idxd_v.at[c1]], add=True)
            return c2

        lax.fori_loop(0, WIN // 2, pair, 0)
        return carry

    lax.fori_loop(0, NWIN, window1, 0)
    plsc.subcore_barrier()

    def out_copy1(j, carry):
        base = sid * RPT + j * CH
        pltpu.sync_copy(acc_sh.at[pl.ds(base, CH), :],
                        agg_out.at[cid, pl.ds(base, CH), :])
        return carry

    lax.fori_loop(0, RCOPIES, out_copy1, 0)

    lax.fori_loop(0, CH, fill_zero, 0)
    lax.fori_loop(0, RCOPIES, zero_acc, 0)
    pltpu.sync_copy(ones_ref, buf_a)
    plsc.subcore_barrier()

    def window2(w, carry):
        pltpu.sync_copy(dst_ref.at[wid, pl.ds(w * WIN, WIN)], idxd_v)

        def cchunk(i, c2):
            pltpu.sync_copy(buf_a, acc_sh.at[idxd_v.at[i]], add=True)
            return c2

        lax.fori_loop(0, WIN, cchunk, 0)
        return carry

    lax.fori_loop(0, NWIN, window2, 0)
    plsc.subcore_barrier()

    def out_copy2(j, carry):
        base = sid * RPT + j * CH
        pltpu.sync_copy(acc_sh.at[pl.ds(base, CH), :],
                        cnt_out.at[cid, pl.ds(base, CH), :])
        return carry

    lax.fori_loop(0, RCOPIES, out_copy2, 0)


@functools.lru_cache(maxsize=1)
def _sc_agg():
    return functools.partial(
        pl.kernel,
        mesh=plsc.VectorSubcoreMesh(core_axis_name="c", subcore_axis_name="s",
                                    num_cores=NC, num_subcores=NS),
        out_type=(
            jax.ShapeDtypeStruct((NC, NPAD, D), jnp.float32),
            jax.ShapeDtypeStruct((NC, NPAD, D), jnp.float32),
        ),
        scratch_types=[
            pltpu.VMEM((WIN, CH), jnp.int32),
            pltpu.VMEM((WIN, CH), jnp.int32),
            pltpu.VMEM((CH, D), jnp.float32),
            pltpu.VMEM((CH, D), jnp.float32),
            pltpu.VMEM_SHARED((NPAD, D), jnp.float32),
            pltpu.SemaphoreType.DMA,
            pltpu.SemaphoreType.DMA,
        ],
    )(_sc_body)


BM = 1000


def _tc_body(p_ref, c_ref, x_ref, w_ref, b_ref, o_ref):
    pa = p_ref[0] + p_ref[1]
    cnt = c_ref[0, :, 0:1] + c_ref[1, :, 0:1]
    inv = jnp.where(cnt > 0, 1.0 / cnt, 0.0)
    h = pa * inv
    dn = (((1,), (1,)), ((), ()))
    out = lax.dot_general(h, w_ref[:, 0:D], dn,
                          preferred_element_type=jnp.float32,
                          precision=lax.Precision.HIGHEST)
    out += lax.dot_general(x_ref[...], w_ref[:, D:2 * D], dn,
                           preferred_element_type=jnp.float32,
                           precision=lax.Precision.HIGHEST)
    o_ref[...] = out + b_ref[...]


def _tc_linear(p, c, x, W, b2):
    return pl.pallas_call(
        _tc_body,
        grid=(N_NODES // BM,),
        in_specs=[
            pl.BlockSpec((NC, BM, D), lambda m: (0, m, 0)),
            pl.BlockSpec((NC, BM, D), lambda m: (0, m, 0)),
            pl.BlockSpec((BM, D), lambda m: (m, 0)),
            pl.BlockSpec((D, 2 * D), lambda m: (0, 0)),
            pl.BlockSpec((1, D), lambda m: (0, 0)),
        ],
        out_specs=pl.BlockSpec((BM, D), lambda m: (m, 0)),
        out_shape=jax.ShapeDtypeStruct((N_NODES, D), jnp.float32),
    )(p, c, x, W, b2)


def kernel(x, edge_index, W, b):
    src = edge_index[0].astype(jnp.int32)
    dst = edge_index[1].astype(jnp.int32)
    pad = EPAD - N_EDGES
    srcp = jnp.concatenate([src, jnp.zeros((pad,), jnp.int32)]
                           ).reshape(NW, NCH, CH)
    dstp = jnp.concatenate([dst, jnp.full((pad,), TRASH, jnp.int32)]
                           ).reshape(NW, NCH, CH)
    onesrow = jnp.zeros((CH, D), jnp.float32).at[:, 0].set(1.0)
    p, c = _sc_agg()(x, srcp, dstp, onesrow)
    b2 = b.reshape(1, D)
    return _tc_linear(p, c, x, W, b2)
